# Initial kernel scaffold; baseline (speedup 1.0000x reference)
import jax
import jax.numpy as jnp
from jax.experimental import pallas as pl


def kernel(list_group_activation, target_labels):
    raise NotImplementedError("write your pallas kernel here")


# speedup vs baseline: 1.6999x; 1.6999x over previous
import functools

import jax
import jax.numpy as jnp
from jax import lax
from jax.experimental import pallas as pl
from jax.experimental.pallas import tpu as pltpu
from jax.experimental.pallas import tpu_sc as plsc

C = 8
N = 8192
G = 128
NC = 2
NS = 16
NW = NC * NS
RPW = N // NW
NCHUNK = 2
CHUNK = RPW // NCHUNK
L = 16


def _sc_body(table, labels, out_f, out_i,
             labels_v, validf_v, idx_v, rows_v, stage_f, stage_i, sem0, sem1):
    cid = lax.axis_index("c")
    sid = lax.axis_index("s")
    wid = cid * NS + sid
    base = wid * RPW

    pltpu.sync_copy(labels.at[pl.ds(base, RPW)], labels_v)

    iota = lax.iota(jnp.int32, L)
    maskv = jnp.zeros((L,), jnp.int32)
    for i in range(RPW // L):
        lab = labels_v[pl.ds(i * L, L)]
        valid = lab >= 1
        clsv = jnp.maximum(lab - 1, 0)
        nvec = base + i * L + iota
        idx = jnp.where(valid, clsv * N + nvec, nvec)
        idx_v[i // (CHUNK // L), pl.ds((i % (CHUNK // L)) * L, L)] = idx
        validf_v[pl.ds(i * L, L)] = jnp.where(valid, 1.0, 0.0).astype(jnp.float32)
        maskv = maskv | jnp.where(valid, jnp.int32(1) << clsv, 0)

    cp0 = pltpu.async_copy(table.at[idx_v.at[0]], rows_v.at[0], sem0)
    cp1 = pltpu.async_copy(table.at[idx_v.at[1]], rows_v.at[1], sem1)

    smax = jnp.float32(0.0)
    smin = jnp.float32(0.0)
    for j in range(NCHUNK):
        (cp0 if j == 0 else cp1).wait()

        def group_body(g, carry, j=j):
            sx, sn = carry
            vfv = validf_v[pl.ds(j * CHUNK + g * L, L)]
            for r in range(L):
                row = g * L + r
                v = rows_v[j, row, pl.ds(0, L)]
                mx = v
                mn = v
                for k in range(1, G // L):
                    v = rows_v[j, row, pl.ds(k * L, L)]
                    mx = jnp.maximum(mx, v)
                    mn = jnp.minimum(mn, v)
                sx = sx + jnp.max(mx) * vfv[r]
                sn = sn + jnp.min(mn) * vfv[r]
            return (sx, sn)

        smax, smin = lax.fori_loop(0, CHUNK // L, group_body, (smax, smin))

    stage_f[...] = jnp.where(iota == 0, smax, smin)
    stage_i[...] = maskv
    pltpu.sync_copy(stage_f, out_f.at[wid])
    pltpu.sync_copy(stage_i, out_i.at[wid])


_sc_minmax = functools.partial(
    pl.kernel,
    out_type=[
        jax.ShapeDtypeStruct((NW, L), jnp.float32),
        jax.ShapeDtypeStruct((NW, L), jnp.int32),
    ],
    mesh=plsc.VectorSubcoreMesh(core_axis_name="c", subcore_axis_name="s"),
    compiler_params=pltpu.CompilerParams(needs_layout_passes=False),
    scratch_types=[
        pltpu.VMEM((RPW,), jnp.int32),
        pltpu.VMEM((RPW,), jnp.float32),
        pltpu.VMEM((NCHUNK, CHUNK), jnp.int32),
        pltpu.VMEM((NCHUNK, CHUNK, G), jnp.float32),
        pltpu.VMEM((L,), jnp.float32),
        pltpu.VMEM((L,), jnp.int32),
        pltpu.SemaphoreType.DMA,
        pltpu.SemaphoreType.DMA,
    ],
)(_sc_body)


def _combine_body(pf_ref, pi_ref, o_ref):
    pf = pf_ref[...]
    pi = pi_ref[...]
    lane = lax.broadcasted_iota(jnp.int32, (NW, L), 1)
    smax = jnp.sum(jnp.where(lane == 0, pf, 0.0))
    smin = jnp.sum(jnp.where(lane == 1, pf, 0.0))
    count = jnp.int32(0)
    for c in range(C):
        count = count + jnp.max((pi >> c) & 1)
    denom = count.astype(jnp.float32) * jnp.float32(N)
    bitpos = lax.broadcasted_iota(jnp.int32, (1, L), 1)
    o_ref[...] = jnp.where(bitpos == 0, -smax / denom, smin / denom)


def kernel(list_group_activation, target_labels):
    table = list_group_activation.reshape(C * N, G)
    pf, pi = _sc_minmax(table, target_labels)
    out = pl.pallas_call(
        _combine_body,
        out_shape=jax.ShapeDtypeStruct((1, L), jnp.float32),
    )(pf, pi)
    return (out[0, 0], out[0, 1])


# speedup vs baseline: 1.8422x; 1.0837x over previous
import functools

import jax
import jax.numpy as jnp
from jax import lax
from jax.experimental import pallas as pl
from jax.experimental.pallas import tpu as pltpu
from jax.experimental.pallas import tpu_sc as plsc

C = 8
N = 8192
G = 128
NC = 2
NS = 16
NW = NC * NS
RPW = N // NW
NCHUNK = 2
CHUNK = RPW // NCHUNK
L = 16
NG = RPW // L
GPC = CHUNK // L


def _sc_body(table, labels, out_f, out_i,
             labels_v, validf_v, idx_v, rows_v, stage_f, stage_i, sem0, sem1):
    cid = lax.axis_index("c")
    sid = lax.axis_index("s")
    wid = cid * NS + sid
    base = wid * RPW

    pltpu.sync_copy(labels.at[pl.ds(base, RPW)], labels_v)

    iota = lax.iota(jnp.int32, L)

    def idx_body(i, maskv):
        lab = labels_v[pl.ds(i * L, L)]
        valid = lab >= 1
        clsv = jnp.maximum(lab - 1, 0)
        nvec = base + i * L + iota
        idx = jnp.where(valid, clsv * N + nvec, nvec)
        idx_v[i // GPC, pl.ds((i % GPC) * L, L)] = idx
        validf_v[pl.ds(i * L, L)] = jnp.where(valid, 1.0, 0.0).astype(jnp.float32)
        return maskv | jnp.where(valid, jnp.int32(1) << clsv, 0)

    maskv = lax.fori_loop(0, NG, idx_body, jnp.zeros((L,), jnp.int32),
                          unroll=1)

    cp0 = pltpu.async_copy(table.at[idx_v.at[0]], rows_v.at[0], sem0)
    cp1 = pltpu.async_copy(table.at[idx_v.at[1]], rows_v.at[1], sem1)

    def group_body(g, carry):
        sx, sn = carry

        @pl.when(g == 0)
        def _():
            cp0.wait()

        @pl.when(g == GPC)
        def _():
            cp1.wait()

        j = g // GPC
        vfv = validf_v[pl.ds(g * L, L)]
        rbase = (g % GPC) * L
        for r in range(L):
            row = rbase + r
            v = rows_v[j, row, pl.ds(0, L)]
            mx = v
            mn = v
            for k in range(1, G // L):
                v = rows_v[j, row, pl.ds(k * L, L)]
                mx = jnp.maximum(mx, v)
                mn = jnp.minimum(mn, v)
            sx = sx + jnp.max(mx) * vfv[r]
            sn = sn + jnp.min(mn) * vfv[r]
        return (sx, sn)

    smax, smin = lax.fori_loop(0, NG, group_body,
                               (jnp.float32(0.0), jnp.float32(0.0)),
                               unroll=1)

    stage_f[...] = jnp.where(iota == 0, smax, smin)
    stage_i[...] = maskv
    pltpu.sync_copy(stage_f, out_f.at[wid])
    pltpu.sync_copy(stage_i, out_i.at[wid])


_sc_minmax = functools.partial(
    pl.kernel,
    out_type=[
        jax.ShapeDtypeStruct((NW, L), jnp.float32),
        jax.ShapeDtypeStruct((NW, L), jnp.int32),
    ],
    mesh=plsc.VectorSubcoreMesh(core_axis_name="c", subcore_axis_name="s"),
    compiler_params=pltpu.CompilerParams(needs_layout_passes=False),
    scratch_types=[
        pltpu.VMEM((RPW,), jnp.int32),
        pltpu.VMEM((RPW,), jnp.float32),
        pltpu.VMEM((NCHUNK, CHUNK), jnp.int32),
        pltpu.VMEM((NCHUNK, CHUNK, G), jnp.float32),
        pltpu.VMEM((L,), jnp.float32),
        pltpu.VMEM((L,), jnp.int32),
        pltpu.SemaphoreType.DMA,
        pltpu.SemaphoreType.DMA,
    ],
)(_sc_body)


def _combine_body(pf_ref, pi_ref, omax_ref, omin_ref):
    pf = pf_ref[...]
    pi = pi_ref[...]
    lane = lax.broadcasted_iota(jnp.int32, (NW, L), 1)
    smax = jnp.sum(jnp.where(lane == 0, pf, 0.0))
    smin = jnp.sum(jnp.where(lane == 1, pf, 0.0))
    count = jnp.int32(0)
    for c in range(C):
        count = count + jnp.max((pi >> c) & 1)
    denom = count.astype(jnp.float32) * jnp.float32(N)
    omax_ref[...] = jnp.full((1, 1), -smax / denom, jnp.float32)
    omin_ref[...] = jnp.full((1, 1), smin / denom, jnp.float32)


def kernel(list_group_activation, target_labels):
    table = list_group_activation.reshape(C * N, G)
    pf, pi = _sc_minmax(table, target_labels)
    omax, omin = pl.pallas_call(
        _combine_body,
        out_shape=[
            jax.ShapeDtypeStruct((1, 1), jnp.float32),
            jax.ShapeDtypeStruct((1, 1), jnp.float32),
        ],
    )(pf, pi)
    return (omax[0, 0], omin[0, 0])
